# baseline (speedup 1.0000x reference)
import functools

import jax
import jax.numpy as jnp
from jax import lax
from jax.experimental import pallas as pl
from jax.experimental.pallas import tpu as pltpu
from jax.experimental.pallas import tpu_sc as plsc

NC = 2
NS = 16
LANES = 128
G = 3


def _ln_k(x, g, b):
    m = x.mean(-1, keepdims=True)
    v = ((x - m) ** 2).mean(-1, keepdims=True)
    return (x - m) * lax.rsqrt(v + 1e-5) * g + b


def _dot_t(a, b):
    return lax.dot_general(a, b, (((1,), (1,)), ((), ())),
                           preferred_element_type=jnp.float32)


def _embed_body(nf_ref, we_ref, be_ref, wv_ref, x_ref, vlo_ref, vhi_ref):
    x = _dot_t(nf_ref[...], we_ref[...]) + be_ref[...]
    x_ref[...] = x
    v = _dot_t(x, wv_ref[...])
    vlo_ref[...] = v[:, :32]
    vhi_ref[...] = v[:, 32:]


def _post_body_v(x_ref, xnlo_ref, xnhi_ref, g_ref, b_ref, w1_ref, b1_ref,
                 w2_ref, b2_ref, wv_ref, xo_ref, vlo_ref, vhi_ref):
    y = x_ref[...] + jnp.concatenate([xnlo_ref[...], xnhi_ref[...]], axis=1)
    y = _ln_k(y, g_ref[...], b_ref[...])
    h = jnp.maximum(_dot_t(y, w1_ref[...]) + b1_ref[...], 0.0)
    z = _ln_k(y + _dot_t(h, w2_ref[...]) + b2_ref[...], g_ref[...], b_ref[...])
    xo_ref[...] = z
    v = _dot_t(z, wv_ref[...])
    vlo_ref[...] = v[:, :32]
    vhi_ref[...] = v[:, 32:]


def _post_body_last(x_ref, xnlo_ref, xnhi_ref, g_ref, b_ref, w1_ref, b1_ref,
                    w2_ref, b2_ref, xo_ref):
    y = x_ref[...] + jnp.concatenate([xnlo_ref[...], xnhi_ref[...]], axis=1)
    y = _ln_k(y, g_ref[...], b_ref[...])
    h = jnp.maximum(_dot_t(y, w1_ref[...]) + b1_ref[...], 0.0)
    z = _ln_k(y + _dot_t(h, w2_ref[...]) + b2_ref[...], g_ref[...], b_ref[...])
    xo_ref[...] = z


def _full(shape):
    return pl.BlockSpec(shape, lambda i: (0,) * len(shape))


def _make_tc_calls(N, DM, DIN, DFF, R):
    grid = (N // R,)
    row2 = pl.BlockSpec((R, DM), lambda i: (i, 0))
    rowh = pl.BlockSpec((R, DM // 2), lambda i: (i, 0))
    f32 = jnp.float32

    embed = pl.pallas_call(
        _embed_body,
        grid=grid,
        in_specs=[pl.BlockSpec((R, DIN), lambda i: (i, 0)),
                  _full((DM, DIN)), _full((1, DM)), _full((DM, DM))],
        out_specs=[row2, rowh, rowh],
        out_shape=[jax.ShapeDtypeStruct((N, DM), f32),
                   jax.ShapeDtypeStruct((N, DM // 2), f32),
                   jax.ShapeDtypeStruct((N, DM // 2), f32)],
    )

    w_specs = [_full((1, DM)), _full((1, DM)), _full((DFF, DM)),
               _full((1, DFF)), _full((DM, DFF)), _full((1, DM))]

    post_v = pl.pallas_call(
        _post_body_v,
        grid=grid,
        in_specs=[row2, rowh, rowh] + w_specs + [_full((DM, DM))],
        out_specs=[row2, rowh, rowh],
        out_shape=[jax.ShapeDtypeStruct((N, DM), f32),
                   jax.ShapeDtypeStruct((N, DM // 2), f32),
                   jax.ShapeDtypeStruct((N, DM // 2), f32)],
    )

    post_last = pl.pallas_call(
        _post_body_last,
        grid=grid,
        in_specs=[row2, rowh, rowh] + w_specs,
        out_specs=row2,
        out_shape=jax.ShapeDtypeStruct((N, DM), f32),
    )
    return embed, post_v, post_last


@functools.lru_cache(maxsize=None)
def _make_sc_scatter(N, EP, DH2):
    rows128 = EP // LANES
    groups = rows128 // (NS * G)
    half = groups // 2
    zspan = ((N + 1 + NS - 1) // NS + 7) // 8 * 8
    acc_rows = NS * zspan
    ospan = (N // NS + 7) // 8 * 8
    olast = N - (NS - 1) * ospan
    f32 = jnp.float32

    mesh = plsc.VectorSubcoreMesh(core_axis_name="c", subcore_axis_name="s",
                                  num_cores=NC, num_subcores=NS)

    @functools.partial(
        pl.kernel,
        mesh=mesh,
        out_type=[jax.ShapeDtypeStruct((N, DH2), f32),
                  jax.ShapeDtypeStruct((N, DH2), f32)],
        scratch_types=[
            pltpu.VMEM((16, LANES), jnp.int32),
            pltpu.VMEM((16, LANES), jnp.int32),
            pltpu.VMEM((G * LANES, DH2), f32),
            pltpu.VMEM((G * LANES, DH2), f32),
            pltpu.VMEM_SHARED((acc_rows, DH2), f32),
            pltpu.SemaphoreType.DMA,
            pltpu.SemaphoreType.DMA,
            pltpu.SemaphoreType.DMA,
        ],
        compiler_params=pltpu.CompilerParams(use_tc_tiling_on_sc=False),
    )
    def sc_scatter(vlo, vhi, srcr, dstr, zs_hbm, outlo, outhi,
                   s_idx, d_idx, rows_a, rows_b, acc, isem, gsem, ssem):
        cid = lax.axis_index("c")
        sid = lax.axis_index("s")

        def run(table, out):
            z0 = pl.multiple_of(sid * zspan, 8)
            pltpu.sync_copy(zs_hbm, acc.at[pl.ds(z0, zspan)])
            plsc.subcore_barrier()
            base = sid * groups * G

            def load_idx(g, buf):
                r0 = base + g * G
                b0 = buf * 8
                a = pltpu.async_copy(srcr.at[pl.ds(r0, G)],
                                     s_idx.at[pl.ds(b0, G)], isem)
                b = pltpu.async_copy(dstr.at[pl.ds(r0, G)],
                                     d_idx.at[pl.ds(b0, G)], isem)
                return a, b

            def gather_group(buf, rows):
                for j in range(G):
                    r0 = j * LANES
                    pltpu.async_copy(table.at[s_idx.at[buf * 8 + j]],
                                     rows.at[pl.ds(r0, LANES)], gsem)
                pltpu.make_async_copy(table.at[pl.ds(0, G * LANES)],
                                      rows, gsem).wait()

            def scatter_group(buf, rows):
                for j in range(G):
                    r0 = j * LANES
                    pltpu.async_copy(rows.at[pl.ds(r0, LANES)],
                                     acc.at[d_idx.at[buf * 8 + j]],
                                     ssem, add=True)

            def drain_scatter(rows):
                pltpu.make_async_copy(table.at[pl.ds(0, G * LANES)],
                                      rows, ssem).wait()

            def gathers_only(buf, rows):
                for j in range(G):
                    pltpu.async_copy(table.at[s_idx.at[buf * 8 + j]],
                                     rows.at[pl.ds(j * LANES, LANES)], gsem)

            def drain_gather(rows):
                pltpu.make_async_copy(table.at[pl.ds(0, G * LANES)],
                                      rows, gsem).wait()

            def body(i, carry):
                ga = 2 * i
                ia = load_idx(ga, 0)
                ib = load_idx(ga + 1, 1)
                for dsc in ia:
                    dsc.wait()
                gathers_only(0, rows_a)

                @pl.when(i > 0)
                def _():
                    drain_scatter(rows_b)

                drain_gather(rows_a)
                scatter_group(0, rows_a)
                for dsc in ib:
                    dsc.wait()
                gathers_only(1, rows_b)
                drain_scatter(rows_a)
                drain_gather(rows_b)
                scatter_group(1, rows_b)
                return carry

            lax.fori_loop(0, half, body, 0)
            drain_scatter(rows_b)
            plsc.subcore_barrier()
            o0 = pl.multiple_of(sid * ospan, 8)

            @pl.when(sid < NS - 1)
            def _():
                pltpu.sync_copy(acc.at[pl.ds(o0, ospan)],
                                out.at[pl.ds(o0, ospan)])

            @pl.when(sid == NS - 1)
            def _():
                ol0 = (NS - 1) * ospan
                pltpu.sync_copy(acc.at[pl.ds(ol0, olast)],
                                out.at[pl.ds(ol0, olast)])

        @pl.when(cid == 0)
        def _():
            run(vlo, outlo)

        @pl.when(cid == 1)
        def _():
            run(vhi, outhi)

    return sc_scatter


def kernel(node_features, edge_index, edge_attr, W_emb, b_emb, Wq, Wk, Wv,
           eW1, eb1, eW2, eb2, ln_g, ln_b, fW1, fb1, fW2, fb2):
    del edge_attr, Wq, Wk, eW1, eb1, eW2, eb2
    N, DIN = node_features.shape
    E = edge_index.shape[1]
    DM = W_emb.shape[0]
    L = Wv.shape[0]
    DFF = fW1.shape[1]
    DH2 = DM // 2

    per_sweep = 2 * NS * G * LANES
    EP = (E + per_sweep - 1) // per_sweep * per_sweep
    src = edge_index[0]
    dst = edge_index[1]
    pad = EP - E
    srcr = jnp.concatenate([src, jnp.zeros((pad,), jnp.int32)]).reshape(
        EP // LANES, LANES)
    dstr = jnp.concatenate([dst, jnp.full((pad,), N, jnp.int32)]).reshape(
        EP // LANES, LANES)

    embed, post_v, post_last = _make_tc_calls(N, DM, DIN, DFF, R=2000)
    sc_scatter = _make_sc_scatter(N, EP, DH2)
    zspan = ((N + 1 + NS - 1) // NS + 7) // 8 * 8
    zs = jnp.zeros((zspan, DH2), jnp.float32)

    b_emb2 = b_emb.reshape(1, DM)
    x, vlo, vhi = embed(node_features, W_emb, b_emb2, Wv[0])
    for l in range(L):
        xnlo, xnhi = sc_scatter(vlo, vhi, srcr, dstr, zs)
        w = (ln_g[l].reshape(1, DM), ln_b[l].reshape(1, DM), fW1[l],
             fb1[l].reshape(1, DFF), fW2[l], fb2[l].reshape(1, DM))
        if l < L - 1:
            x, vlo, vhi = post_v(x, xnlo, xnhi, *w, Wv[l + 1])
        else:
            x = post_last(x, xnlo, xnhi, *w)
    return x


# speedup vs baseline: 1.0004x; 1.0004x over previous
import functools

import jax
import jax.numpy as jnp
from jax import lax
from jax.experimental import pallas as pl
from jax.experimental.pallas import tpu as pltpu
from jax.experimental.pallas import tpu_sc as plsc

NC = 2
NS = 16
LANES = 128
G = 3


def _ln_k(x, g, b):
    m = x.mean(-1, keepdims=True)
    v = ((x - m) ** 2).mean(-1, keepdims=True)
    return (x - m) * lax.rsqrt(v + 1e-5) * g + b


def _dot_t(a, b):
    return lax.dot_general(a, b, (((1,), (1,)), ((), ())),
                           preferred_element_type=jnp.float32)


def _embed_body(nf_ref, we_ref, be_ref, wv_ref, x_ref, vlo_ref, vhi_ref):
    x = _dot_t(nf_ref[...], we_ref[...]) + be_ref[...]
    x_ref[...] = x
    v = _dot_t(x, wv_ref[...])
    vlo_ref[...] = v[:, :32]
    vhi_ref[...] = v[:, 32:]


def _post_body_v(x_ref, xnlo_ref, xnhi_ref, g_ref, b_ref, w1_ref, b1_ref,
                 w2_ref, b2_ref, wv_ref, xo_ref, vlo_ref, vhi_ref):
    y = x_ref[...] + jnp.concatenate([xnlo_ref[...], xnhi_ref[...]], axis=1)
    y = _ln_k(y, g_ref[...], b_ref[...])
    h = jnp.maximum(_dot_t(y, w1_ref[...]) + b1_ref[...], 0.0)
    z = _ln_k(y + _dot_t(h, w2_ref[...]) + b2_ref[...], g_ref[...], b_ref[...])
    xo_ref[...] = z
    v = _dot_t(z, wv_ref[...])
    vlo_ref[...] = v[:, :32]
    vhi_ref[...] = v[:, 32:]


def _post_body_last(x_ref, xnlo_ref, xnhi_ref, g_ref, b_ref, w1_ref, b1_ref,
                    w2_ref, b2_ref, xo_ref):
    y = x_ref[...] + jnp.concatenate([xnlo_ref[...], xnhi_ref[...]], axis=1)
    y = _ln_k(y, g_ref[...], b_ref[...])
    h = jnp.maximum(_dot_t(y, w1_ref[...]) + b1_ref[...], 0.0)
    z = _ln_k(y + _dot_t(h, w2_ref[...]) + b2_ref[...], g_ref[...], b_ref[...])
    xo_ref[...] = z


def _full(shape):
    return pl.BlockSpec(shape, lambda i: (0,) * len(shape))


def _make_tc_calls(N, DM, DIN, DFF, R):
    grid = (N // R,)
    row2 = pl.BlockSpec((R, DM), lambda i: (i, 0))
    rowh = pl.BlockSpec((R, DM // 2), lambda i: (i, 0))
    f32 = jnp.float32

    embed = pl.pallas_call(
        _embed_body,
        grid=grid,
        in_specs=[pl.BlockSpec((R, DIN), lambda i: (i, 0)),
                  _full((DM, DIN)), _full((1, DM)), _full((DM, DM))],
        out_specs=[row2, rowh, rowh],
        out_shape=[jax.ShapeDtypeStruct((N, DM), f32),
                   jax.ShapeDtypeStruct((N, DM // 2), f32),
                   jax.ShapeDtypeStruct((N, DM // 2), f32)],
    )

    w_specs = [_full((1, DM)), _full((1, DM)), _full((DFF, DM)),
               _full((1, DFF)), _full((DM, DFF)), _full((1, DM))]

    post_v = pl.pallas_call(
        _post_body_v,
        grid=grid,
        in_specs=[row2, rowh, rowh] + w_specs + [_full((DM, DM))],
        out_specs=[row2, rowh, rowh],
        out_shape=[jax.ShapeDtypeStruct((N, DM), f32),
                   jax.ShapeDtypeStruct((N, DM // 2), f32),
                   jax.ShapeDtypeStruct((N, DM // 2), f32)],
    )

    post_last = pl.pallas_call(
        _post_body_last,
        grid=grid,
        in_specs=[row2, rowh, rowh] + w_specs,
        out_specs=row2,
        out_shape=jax.ShapeDtypeStruct((N, DM), f32),
    )
    return embed, post_v, post_last


@functools.lru_cache(maxsize=None)
def _make_sc_scatter(N, EP, DH2):
    rows128 = EP // LANES
    groups = rows128 // (NS * G)
    half = groups // 2
    zspan = ((N + 1 + NS - 1) // NS + 7) // 8 * 8
    acc_rows = NS * zspan
    ospan = (N // NS + 7) // 8 * 8
    olast = N - (NS - 1) * ospan
    f32 = jnp.float32

    mesh = plsc.VectorSubcoreMesh(core_axis_name="c", subcore_axis_name="s",
                                  num_cores=NC, num_subcores=NS)

    @functools.partial(
        pl.kernel,
        mesh=mesh,
        out_type=[jax.ShapeDtypeStruct((N, DH2), f32),
                  jax.ShapeDtypeStruct((N, DH2), f32)],
        scratch_types=[
            pltpu.VMEM((16, LANES), jnp.int32),
            pltpu.VMEM((16, LANES), jnp.int32),
            pltpu.VMEM((G * LANES, DH2), f32),
            pltpu.VMEM((G * LANES, DH2), f32),
            pltpu.VMEM_SHARED((acc_rows, DH2), f32),
            pltpu.SemaphoreType.DMA,
            pltpu.SemaphoreType.DMA,
            pltpu.SemaphoreType.DMA,
        ],
        compiler_params=pltpu.CompilerParams(use_tc_tiling_on_sc=False),
    )
    def sc_scatter(vlo, vhi, srcr, dstr, zs_hbm, outlo, outhi,
                   s_idx, d_idx, rows_a, rows_b, acc, isem, gsem, ssem):
        cid = lax.axis_index("c")
        sid = lax.axis_index("s")

        def run(table, out):
            z0 = pl.multiple_of(sid * zspan, 8)
            pltpu.sync_copy(zs_hbm, acc.at[pl.ds(z0, zspan)])
            plsc.subcore_barrier()
            base = sid * groups * G

            def load_idx(g, buf):
                r0 = base + g * G
                b0 = buf * 8
                a = pltpu.async_copy(srcr.at[pl.ds(r0, G)],
                                     s_idx.at[pl.ds(b0, G)], isem)
                b = pltpu.async_copy(dstr.at[pl.ds(r0, G)],
                                     d_idx.at[pl.ds(b0, G)], isem)
                return a, b

            def gather_group(buf, rows):
                for j in range(G):
                    r0 = j * LANES
                    pltpu.async_copy(table.at[s_idx.at[buf * 8 + j]],
                                     rows.at[pl.ds(r0, LANES)], gsem)
                pltpu.make_async_copy(table.at[pl.ds(0, G * LANES)],
                                      rows, gsem).wait()

            def scatter_group(buf, rows):
                for j in range(G):
                    r0 = j * LANES
                    pltpu.async_copy(rows.at[pl.ds(r0, LANES)],
                                     acc.at[d_idx.at[buf * 8 + j]],
                                     ssem, add=True)

            def drain_scatter(rows):
                pltpu.make_async_copy(table.at[pl.ds(0, G * LANES)],
                                      rows, ssem).wait()

            def gathers_only(buf, rows):
                for j in range(G):
                    pltpu.async_copy(table.at[s_idx.at[buf * 8 + j]],
                                     rows.at[pl.ds(j * LANES, LANES)], gsem)

            def drain_gather(rows):
                pltpu.make_async_copy(table.at[pl.ds(0, G * LANES)],
                                      rows, gsem).wait()

            def body(i, carry):
                ga = 2 * i
                ia = load_idx(ga, 0)
                ib = load_idx(ga + 1, 1)
                for dsc in ia:
                    dsc.wait()
                gathers_only(0, rows_a)

                @pl.when(i > 0)
                def _():
                    drain_scatter(rows_b)

                drain_gather(rows_a)
                scatter_group(0, rows_a)
                for dsc in ib:
                    dsc.wait()
                gathers_only(1, rows_b)
                drain_scatter(rows_a)
                drain_gather(rows_b)
                scatter_group(1, rows_b)
                return carry

            lax.fori_loop(0, half, body, 0)
            drain_scatter(rows_b)
            plsc.subcore_barrier()
            o0 = pl.multiple_of(sid * ospan, 8)

            @pl.when(sid < NS - 1)
            def _():
                pltpu.sync_copy(acc.at[pl.ds(o0, ospan)],
                                out.at[pl.ds(o0, ospan)])

            @pl.when(sid == NS - 1)
            def _():
                ol0 = (NS - 1) * ospan
                pltpu.sync_copy(acc.at[pl.ds(ol0, olast)],
                                out.at[pl.ds(ol0, olast)])

        @pl.when(cid == 0)
        def _():
            run(vlo, outlo)

        @pl.when(cid == 1)
        def _():
            run(vhi, outhi)

    return sc_scatter


def kernel(node_features, edge_index, edge_attr, W_emb, b_emb, Wq, Wk, Wv,
           eW1, eb1, eW2, eb2, ln_g, ln_b, fW1, fb1, fW2, fb2):
    del edge_attr, Wq, Wk, eW1, eb1, eW2, eb2
    N, DIN = node_features.shape
    E = edge_index.shape[1]
    DM = W_emb.shape[0]
    L = Wv.shape[0]
    DFF = fW1.shape[1]
    DH2 = DM // 2

    per_sweep = 2 * NS * G * LANES
    EP = (E + per_sweep - 1) // per_sweep * per_sweep
    src = edge_index[0]
    dst = edge_index[1]
    pad = EP - E
    zspan = ((N + 1 + NS - 1) // NS + 7) // 8 * 8
    n_trash = NS * zspan - N
    trash = N + jnp.arange(pad, dtype=jnp.int32) % n_trash
    srcr = jnp.concatenate([src, jnp.zeros((pad,), jnp.int32)]).reshape(
        EP // LANES, LANES)
    dstr = jnp.concatenate([dst, trash]).reshape(EP // LANES, LANES)

    embed, post_v, post_last = _make_tc_calls(N, DM, DIN, DFF, R=2000)
    sc_scatter = _make_sc_scatter(N, EP, DH2)
    zs = jnp.zeros((zspan, DH2), jnp.float32)

    b_emb2 = b_emb.reshape(1, DM)
    x, vlo, vhi = embed(node_features, W_emb, b_emb2, Wv[0])
    for l in range(L):
        xnlo, xnhi = sc_scatter(vlo, vhi, srcr, dstr, zs)
        w = (ln_g[l].reshape(1, DM), ln_b[l].reshape(1, DM), fW1[l],
             fb1[l].reshape(1, DFF), fW2[l], fb2[l].reshape(1, DM))
        if l < L - 1:
            x, vlo, vhi = post_v(x, xnlo, xnhi, *w, Wv[l + 1])
        else:
            x = post_last(x, xnlo, xnhi, *w)
    return x


# speedup vs baseline: 1.2068x; 1.2063x over previous
import functools

import jax
import jax.numpy as jnp
from jax import lax
from jax.experimental import pallas as pl
from jax.experimental.pallas import tpu as pltpu
from jax.experimental.pallas import tpu_sc as plsc

NC = 2
NS = 16
LANES = 128
G = 3


def _ln_k(x, g, b):
    m = x.mean(-1, keepdims=True)
    v = ((x - m) ** 2).mean(-1, keepdims=True)
    return (x - m) * lax.rsqrt(v + 1e-5) * g + b


def _dot_t(a, b):
    return lax.dot_general(a, b, (((1,), (1,)), ((), ())),
                           preferred_element_type=jnp.float32)


def _embed_body(nf_ref, we_ref, be_ref, wv_ref, x_ref, vlo_ref, vhi_ref):
    x = _dot_t(nf_ref[...], we_ref[...]) + be_ref[...]
    x_ref[...] = x
    v = _dot_t(x, wv_ref[...])
    vlo_ref[...] = v[:, :32]
    vhi_ref[...] = v[:, 32:]


def _post_body_v(x_ref, xnlo_ref, xnhi_ref, g_ref, b_ref, w1_ref, b1_ref,
                 w2_ref, b2_ref, wv_ref, xo_ref, vlo_ref, vhi_ref):
    y = x_ref[...] + jnp.concatenate([xnlo_ref[...], xnhi_ref[...]], axis=1)
    y = _ln_k(y, g_ref[...], b_ref[...])
    h = jnp.maximum(_dot_t(y, w1_ref[...]) + b1_ref[...], 0.0)
    z = _ln_k(y + _dot_t(h, w2_ref[...]) + b2_ref[...], g_ref[...], b_ref[...])
    xo_ref[...] = z
    v = _dot_t(z, wv_ref[...])
    vlo_ref[...] = v[:, :32]
    vhi_ref[...] = v[:, 32:]


def _post_body_last(x_ref, xnlo_ref, xnhi_ref, g_ref, b_ref, w1_ref, b1_ref,
                    w2_ref, b2_ref, xo_ref):
    y = x_ref[...] + jnp.concatenate([xnlo_ref[...], xnhi_ref[...]], axis=1)
    y = _ln_k(y, g_ref[...], b_ref[...])
    h = jnp.maximum(_dot_t(y, w1_ref[...]) + b1_ref[...], 0.0)
    z = _ln_k(y + _dot_t(h, w2_ref[...]) + b2_ref[...], g_ref[...], b_ref[...])
    xo_ref[...] = z


def _full(shape):
    return pl.BlockSpec(shape, lambda i: (0,) * len(shape))


def _make_tc_calls(N, DM, DIN, DFF, R):
    grid = (N // R,)
    row2 = pl.BlockSpec((R, DM), lambda i: (i, 0))
    rowh = pl.BlockSpec((R, DM // 2), lambda i: (i, 0))
    f32 = jnp.float32

    embed = pl.pallas_call(
        _embed_body,
        grid=grid,
        in_specs=[pl.BlockSpec((R, DIN), lambda i: (i, 0)),
                  _full((DM, DIN)), _full((1, DM)), _full((DM, DM))],
        out_specs=[row2, rowh, rowh],
        out_shape=[jax.ShapeDtypeStruct((N, DM), f32),
                   jax.ShapeDtypeStruct((N, DM // 2), f32),
                   jax.ShapeDtypeStruct((N, DM // 2), f32)],
    )

    w_specs = [_full((1, DM)), _full((1, DM)), _full((DFF, DM)),
               _full((1, DFF)), _full((DM, DFF)), _full((1, DM))]

    post_v = pl.pallas_call(
        _post_body_v,
        grid=grid,
        in_specs=[row2, rowh, rowh] + w_specs + [_full((DM, DM))],
        out_specs=[row2, rowh, rowh],
        out_shape=[jax.ShapeDtypeStruct((N, DM), f32),
                   jax.ShapeDtypeStruct((N, DM // 2), f32),
                   jax.ShapeDtypeStruct((N, DM // 2), f32)],
    )

    post_last = pl.pallas_call(
        _post_body_last,
        grid=grid,
        in_specs=[row2, rowh, rowh] + w_specs,
        out_specs=row2,
        out_shape=jax.ShapeDtypeStruct((N, DM), f32),
    )
    return embed, post_v, post_last


@functools.lru_cache(maxsize=None)
def _make_sc_scatter(N, EP, DH2):
    rows128 = EP // LANES
    groups = rows128 // (NS * G)
    half = groups // 2
    zspan = ((N + 1 + NS - 1) // NS + 7) // 8 * 8
    acc_rows = NS * zspan
    ospan = (N // NS + 7) // 8 * 8
    olast = N - (NS - 1) * ospan
    f32 = jnp.float32

    mesh = plsc.VectorSubcoreMesh(core_axis_name="c", subcore_axis_name="s",
                                  num_cores=NC, num_subcores=NS)

    @functools.partial(
        pl.kernel,
        mesh=mesh,
        out_type=[jax.ShapeDtypeStruct((N, DH2), f32),
                  jax.ShapeDtypeStruct((N, DH2), f32)],
        scratch_types=[
            pltpu.VMEM((16, LANES), jnp.int32),
            pltpu.VMEM((16, LANES), jnp.int32),
            pltpu.VMEM((G * LANES, DH2), f32),
            pltpu.VMEM((G * LANES, DH2), f32),
            pltpu.VMEM_SHARED((acc_rows, DH2), f32),
            pltpu.SemaphoreType.DMA,
            pltpu.SemaphoreType.DMA,
            pltpu.SemaphoreType.DMA,
        ],
        compiler_params=pltpu.CompilerParams(use_tc_tiling_on_sc=False),
    )
    def sc_scatter(vlo, vhi, srcr, dstr, zs_hbm, outlo, outhi,
                   s_idx, d_idx, rows_a, rows_b, acc, isem, gsem, ssem):
        cid = lax.axis_index("c")
        sid = lax.axis_index("s")

        def run(table, out):
            z0 = pl.multiple_of(sid * zspan, 8)
            pltpu.sync_copy(zs_hbm, acc.at[pl.ds(z0, zspan)])
            plsc.subcore_barrier()
            base = sid * groups * G

            def load_idx(g, buf):
                r0 = base + g * G
                b0 = buf * 8
                a = pltpu.async_copy(srcr.at[pl.ds(r0, G)],
                                     s_idx.at[pl.ds(b0, G)], isem)
                b = pltpu.async_copy(dstr.at[pl.ds(r0, G)],
                                     d_idx.at[pl.ds(b0, G)], isem)
                return a, b

            def gather_group(buf, rows):
                for j in range(G):
                    r0 = j * LANES
                    pltpu.async_copy(table.at[s_idx.at[buf * 8 + j]],
                                     rows.at[pl.ds(r0, LANES)], gsem)
                pltpu.make_async_copy(table.at[pl.ds(0, G * LANES)],
                                      rows, gsem).wait()

            def scatter_group(buf, rows):
                for j in range(G):
                    r0 = j * LANES
                    pltpu.async_copy(rows.at[pl.ds(r0, LANES)],
                                     acc.at[d_idx.at[buf * 8 + j]],
                                     ssem, add=True)

            def drain_scatter(rows):
                pltpu.make_async_copy(table.at[pl.ds(0, G * LANES)],
                                      rows, ssem).wait()

            def gathers_only(buf, rows):
                for j in range(G):
                    pltpu.async_copy(table.at[s_idx.at[buf * 8 + j]],
                                     rows.at[pl.ds(j * LANES, LANES)], gsem)

            def drain_gather(rows):
                pltpu.make_async_copy(table.at[pl.ds(0, G * LANES)],
                                      rows, gsem).wait()

            def body(i, carry):
                ga = 2 * i
                ia = load_idx(ga, 0)
                ib = load_idx(ga + 1, 1)
                for dsc in ia:
                    dsc.wait()
                gathers_only(0, rows_a)

                @pl.when(i > 0)
                def _():
                    drain_scatter(rows_b)

                drain_gather(rows_a)
                scatter_group(0, rows_a)
                for dsc in ib:
                    dsc.wait()
                gathers_only(1, rows_b)
                drain_scatter(rows_a)
                drain_gather(rows_b)
                scatter_group(1, rows_b)
                return carry

            lax.fori_loop(0, half, body, 0)
            drain_scatter(rows_b)
            for g in range(2 * half, groups):
                ia = load_idx(g, 0)
                for dsc in ia:
                    dsc.wait()
                gathers_only(0, rows_a)
                drain_gather(rows_a)
                scatter_group(0, rows_a)
                drain_scatter(rows_a)
            plsc.subcore_barrier()
            o0 = pl.multiple_of(sid * ospan, 8)

            @pl.when(sid < NS - 1)
            def _():
                pltpu.sync_copy(acc.at[pl.ds(o0, ospan)],
                                out.at[pl.ds(o0, ospan)])

            @pl.when(sid == NS - 1)
            def _():
                ol0 = (NS - 1) * ospan
                pltpu.sync_copy(acc.at[pl.ds(ol0, olast)],
                                out.at[pl.ds(ol0, olast)])

        @pl.when(cid == 0)
        def _():
            run(vlo, outlo)

        @pl.when(cid == 1)
        def _():
            run(vhi, outhi)

    return sc_scatter


def kernel(node_features, edge_index, edge_attr, W_emb, b_emb, Wq, Wk, Wv,
           eW1, eb1, eW2, eb2, ln_g, ln_b, fW1, fb1, fW2, fb2):
    del edge_attr, Wq, Wk, eW1, eb1, eW2, eb2
    N, DIN = node_features.shape
    E = edge_index.shape[1]
    DM = W_emb.shape[0]
    L = Wv.shape[0]
    DFF = fW1.shape[1]
    DH2 = DM // 2

    per_sweep = NS * G * LANES
    EP = (E + per_sweep - 1) // per_sweep * per_sweep
    src = edge_index[0]
    dst = edge_index[1]
    pad = EP - E
    zspan = ((N + 1 + NS - 1) // NS + 7) // 8 * 8
    n_trash = NS * zspan - N
    trash = N + jnp.arange(pad, dtype=jnp.int32) % n_trash
    srcr = jnp.concatenate([src, jnp.zeros((pad,), jnp.int32)]).reshape(
        EP // LANES, LANES)
    dstr = jnp.concatenate([dst, trash]).reshape(EP // LANES, LANES)

    embed, post_v, post_last = _make_tc_calls(N, DM, DIN, DFF, R=2000)
    sc_scatter = _make_sc_scatter(N, EP, DH2)
    zs = jnp.zeros((zspan, DH2), jnp.float32)

    b_emb2 = b_emb.reshape(1, DM)
    x, vlo, vhi = embed(node_features, W_emb, b_emb2, Wv[0])
    for l in range(L):
        xnlo, xnhi = sc_scatter(vlo, vhi, srcr, dstr, zs)
        w = (ln_g[l].reshape(1, DM), ln_b[l].reshape(1, DM), fW1[l],
             fb1[l].reshape(1, DFF), fW2[l], fb2[l].reshape(1, DM))
        if l < L - 1:
            x, vlo, vhi = post_v(x, xnlo, xnhi, *w, Wv[l + 1])
        else:
            x = post_last(x, xnlo, xnhi, *w)
    return x
